# baseline (speedup 1.0000x reference)
import functools

import jax
import jax.numpy as jnp
from jax import lax
from jax.experimental import pallas as pl
from jax.experimental.pallas import tpu as pltpu
from jax.experimental.pallas import tpu_sc as plsc

NUM_CLASS = 100000
CTX_DIM = 512
N_CLS_CTX = 4
N_PRE = 5
N_SUF = 68
CLIP_LEN = 77
BATCH = 4096

NC = 2
NS = 16
NW = NC * NS
BPW = BATCH // NW
K = 16


@functools.partial(
    pl.kernel,
    mesh=plsc.VectorSubcoreMesh(core_axis_name="c", subcore_axis_name="s"),
    out_type=jax.ShapeDtypeStruct((BATCH, CLIP_LEN, CTX_DIM), jnp.float32),
    scratch_types=[
        pltpu.VMEM((BPW,), jnp.int32),
        pltpu.VMEM((K, N_CLS_CTX, CTX_DIM), jnp.float32),
        pltpu.VMEM((1, CLIP_LEN, CTX_DIM), jnp.float32),
        pltpu.SemaphoreType.DMA,
    ],
    compiler_params=pltpu.CompilerParams(use_tc_tiling_on_sc=False),
)
def _prompt_assemble(label_h, cls_h, pre_h, suf_h, out_h, idx_v, cls_v, blk, gsem):
    cid = lax.axis_index("c")
    sid = lax.axis_index("s")
    wid = sid * NC + cid
    base = wid * BPW

    pltpu.sync_copy(label_h.at[pl.ds(base, BPW)], idx_v)
    pltpu.sync_copy(pre_h, blk.at[:, pl.ds(0, N_PRE)])
    pltpu.sync_copy(suf_h, blk.at[:, pl.ds(N_PRE + N_CLS_CTX, N_SUF)])

    def chunk(c, _):
        pltpu.async_copy(cls_h.at[idx_v.at[pl.ds(c * K, K)]], cls_v, gsem).wait()

        def elem(j, _):
            def splice_row(r, _):
                for v in range(CTX_DIM // 16):
                    blk[0, N_PRE + r, pl.ds(v * 16, 16)] = cls_v[j, r, pl.ds(v * 16, 16)]
                return None

            lax.fori_loop(0, N_CLS_CTX, splice_row, None)
            pltpu.sync_copy(blk, out_h.at[pl.ds(base + c * K + j, 1)])
            return None

        lax.fori_loop(0, K, elem, None)
        return None

    lax.fori_loop(0, BPW // K, chunk, None)


def kernel(label, cls_ctx, token_prefix, token_suffix):
    return _prompt_assemble(
        label.astype(jnp.int32),
        cls_ctx,
        token_prefix,
        token_suffix,
    )
